# Initial kernel scaffold; baseline (speedup 1.0000x reference)
import jax
import jax.numpy as jnp
from jax.experimental import pallas as pl


def kernel(mask, bbox):
    raise NotImplementedError("write your pallas kernel here")


# speedup vs baseline: 76.0765x; 76.0765x over previous
import jax
import jax.numpy as jnp
import numpy as np
from jax import lax
from jax.experimental import pallas as pl
from jax.experimental.pallas import tpu as pltpu

_NUM_BINS = 36
_MIN_ACTIVATION = 0.1
_PENALTY_WEIGHT = 1.0
_H = 384
_W = 384
_B = 64


def _loss_body(scal_ref, m_ref, out_ref):
    b = pl.program_id(0)
    m = m_ref[0]
    cx = scal_ref[0, b]
    cy = scal_ref[1, b]
    x = lax.broadcasted_iota(jnp.int32, (_H, _W), 1).astype(jnp.float32)
    y = lax.broadcasted_iota(jnp.int32, (_H, _W), 0).astype(jnp.float32)
    ang = jnp.arctan2(y - cy, x - cx)
    an = (ang + np.pi) / (2 * np.pi) * _NUM_BINS
    bins = jnp.clip(an.astype(jnp.int32), 0, _NUM_BINS - 1)

    under = jnp.float32(0.0)
    for k in range(_NUM_BINS):
        indf = (bins == k).astype(jnp.float32)
        s = jnp.sum(indf * m)
        c = jnp.sum(indf)
        act = jnp.where(c > 0, s / jnp.maximum(c, 1.0), 0.0)
        under += (act < _MIN_ACTIVATION).astype(jnp.float32)
    pen = under / jnp.float32(_NUM_BINS)

    acc = jnp.where(b == 0, 0.0, out_ref[0, 0]) + pen
    acc = jnp.where(b == _B - 1, _PENALTY_WEIGHT * acc / jnp.float32(_B), acc)
    out_ref[0, 0] = acc


def kernel(mask, bbox):
    m = mask.reshape(_B, _H, _W)
    cx = bbox[:, 0] * _W
    cy = bbox[:, 1] * _H
    scal = jnp.stack([cx, cy])
    out = pl.pallas_call(
        _loss_body,
        grid=(_B,),
        in_specs=[
            pl.BlockSpec(memory_space=pltpu.SMEM),
            pl.BlockSpec((1, _H, _W), lambda b: (b, 0, 0)),
        ],
        out_specs=pl.BlockSpec(memory_space=pltpu.SMEM),
        out_shape=jax.ShapeDtypeStruct((1, 1), jnp.float32),
    )(scal, m)
    return out[0, 0]
